# baseline (speedup 1.0000x reference)
import functools

import jax
import jax.numpy as jnp
from jax import lax
from jax.experimental import pallas as pl
from jax.experimental.pallas import tpu as pltpu
from jax.experimental.pallas import tpu_sc as plsc

B = 1024
L = 200
DIM = 128
EPS = 1e-12
N = B * L

_info = plsc.get_sparse_core_info()
NC, NS = _info.num_cores, _info.num_subcores
NW = NC * NS
TOK_W = N // NW
C = 64
NCHUNK = TOK_W // C
NBUF = 4
NGROUP = NCHUNK // NBUF
NREG = DIM // 16


def _rsqrt(x):
    i = lax.bitcast_convert_type(x, jnp.int32)
    i = jnp.int32(0x5F3759DF) - (i >> 1)
    y = lax.bitcast_convert_type(i, jnp.float32)
    for _ in range(3):
        y = y * (1.5 - 0.5 * x * y * y)
    return y


def _tree_sum(vs):
    while len(vs) > 1:
        vs = [a + b for a, b in zip(vs[::2], vs[1::2])]
    return vs[0]


def _sc_body(ids_hbm, word_hbm, pos_hbm, gamma_hbm, beta_hbm,
             normed_hbm, words_hbm, ids_v, pos_v, gamma_v, beta_v, *scr):
    rows = scr[:NBUF]
    norm = scr[NBUF:2 * NBUF]
    gsem = scr[2 * NBUF:3 * NBUF]
    wsem = scr[3 * NBUF:4 * NBUF]
    nsem = scr[4 * NBUF:5 * NBUF]

    wid = lax.axis_index("s") * NC + lax.axis_index("c")
    base = wid * TOK_W

    pltpu.sync_copy(ids_hbm.at[pl.ds(base, TOK_W)], ids_v)
    pltpu.sync_copy(pos_hbm.at[pl.ds(0, L)], pos_v)
    pltpu.sync_copy(gamma_hbm, gamma_v)
    pltpu.sync_copy(beta_hbm, beta_v)

    gamma_regs = [gamma_v[pl.ds(16 * j, 16)] for j in range(NREG)]
    beta_regs = [beta_v[pl.ds(16 * j, 16)] for j in range(NREG)]

    def gather(c, b):
        return pltpu.make_async_copy(
            word_hbm.at[ids_v.at[pl.ds(c * C, C)]], rows[b], gsem[b])

    def words_out(c, b):
        return pltpu.make_async_copy(
            rows[b], words_hbm.at[pl.ds(base + c * C, C)], wsem[b])

    def norm_out(c, b):
        return pltpu.make_async_copy(
            norm[b], normed_hbm.at[pl.ds(base + c * C, C)], nsem[b])

    for b in range(NBUF):
        gather(b, b).start()

    def compute_chunk(c, b):
        rows_v, norm_v = rows[b], norm[b]
        off = lax.rem(c * C, L)

        @plsc.parallel_loop(0, C, unroll=4)
        def _tok(t):
            p = off + t
            p = jnp.where(p >= L, p - L, p)
            x = [rows_v[t, pl.ds(16 * j, 16)] + pos_v[p, pl.ds(16 * j, 16)]
                 for j in range(NREG)]
            s = jnp.sum(_tree_sum(x))
            q = jnp.sum(_tree_sum([xj * xj for xj in x]))
            mean = s * (1.0 / DIM)
            var = q * (1.0 / DIM) - mean * mean
            r = _rsqrt(var + EPS)
            a = [r * g for g in gamma_regs]
            for j in range(NREG):
                norm_v[t, pl.ds(16 * j, 16)] = (
                    (x[j] - mean) * a[j] + beta_regs[j])

    def group_body(g, carry):
        for b in range(NBUF):
            c = g * NBUF + b
            gather(c, b).wait()
            words_out(c, b).start()

            @pl.when(g >= 1)
            def _():
                norm_out(c - NBUF, b).wait()

            compute_chunk(c, b)
            norm_out(c, b).start()

            @pl.when(g < NGROUP - 1)
            def _():
                words_out(c, b).wait()
                gather(c + NBUF, b).start()
        return carry

    lax.fori_loop(0, NGROUP, group_body, 0)

    for b in range(NBUF):
        words_out(NCHUNK - NBUF + b, b).wait()
        norm_out(NCHUNK - NBUF + b, b).wait()


@functools.partial(jax.jit, static_argnames=())
def kernel(input_ids, word_table, pos_table, gamma, beta):
    ids_flat = input_ids.reshape(N).astype(jnp.int32)
    mesh = plsc.VectorSubcoreMesh(core_axis_name="c", subcore_axis_name="s")
    normed, words = pl.kernel(
        _sc_body,
        out_type=[
            jax.ShapeDtypeStruct((N, DIM), jnp.float32),
            jax.ShapeDtypeStruct((N, DIM), jnp.float32),
        ],
        mesh=mesh,
        compiler_params=pltpu.CompilerParams(needs_layout_passes=False),
        scratch_types=(
            [
                pltpu.VMEM((TOK_W,), jnp.int32),
                pltpu.VMEM((L, DIM), jnp.float32),
                pltpu.VMEM((DIM,), jnp.float32),
                pltpu.VMEM((DIM,), jnp.float32),
            ]
            + [pltpu.VMEM((C, DIM), jnp.float32)] * (2 * NBUF)
            + [pltpu.SemaphoreType.DMA] * (3 * NBUF)
        ),
    )(ids_flat, word_table, pos_table, gamma, beta)
    return (normed.reshape(B, L, DIM), words.reshape(B, L, DIM))


# speedup vs baseline: 1.3355x; 1.3355x over previous
import functools

import jax
import jax.numpy as jnp
from jax import lax
from jax.experimental import pallas as pl
from jax.experimental.pallas import tpu as pltpu
from jax.experimental.pallas import tpu_sc as plsc

B = 1024
L = 200
DIM = 128
EPS = 1e-12
N = B * L

_info = plsc.get_sparse_core_info()
NC, NS = _info.num_cores, _info.num_subcores
NW = NC * NS
TOK_W = N // NW
C = 64
NCHUNK = TOK_W // C
NBUF = 4
NGROUP = NCHUNK // NBUF
NREG = DIM // 16


def _rsqrt(x):
    i = lax.bitcast_convert_type(x, jnp.int32)
    i = jnp.int32(0x5F3759DF) - (i >> 1)
    y = lax.bitcast_convert_type(i, jnp.float32)
    for _ in range(3):
        y = y * (1.5 - 0.5 * x * y * y)
    return y


def _tree_sum(vs):
    while len(vs) > 1:
        vs = [a + b for a, b in zip(vs[::2], vs[1::2])]
    return vs[0]


def _sc_body(ids_hbm, word_hbm, pos_hbm, gamma_hbm, beta_hbm,
             normed_hbm, words_hbm, ids_v, pos_v, gamma_v, beta_v, *scr):
    rows = scr[:NBUF]
    norm = scr[NBUF:2 * NBUF]
    gsem = scr[2 * NBUF:3 * NBUF]
    wsem = scr[3 * NBUF:4 * NBUF]
    nsem = scr[4 * NBUF:5 * NBUF]

    wid = lax.axis_index("s") * NC + lax.axis_index("c")
    base = wid * TOK_W

    ids_cp = pltpu.async_copy(ids_hbm.at[pl.ds(base, TOK_W)], ids_v, nsem[0])
    pos_cp = pltpu.async_copy(pos_hbm.at[pl.ds(0, L)], pos_v, nsem[1])
    g_cp = pltpu.async_copy(gamma_hbm, gamma_v, nsem[2])
    b_cp = pltpu.async_copy(beta_hbm, beta_v, nsem[3])
    ids_cp.wait()

    def gather(c, b):
        return pltpu.make_async_copy(
            word_hbm.at[ids_v.at[pl.ds(c * C, C)]], rows[b], gsem[b])

    def words_out(c, b):
        return pltpu.make_async_copy(
            rows[b], words_hbm.at[pl.ds(base + c * C, C)], wsem[b])

    def norm_out(c, b):
        return pltpu.make_async_copy(
            norm[b], normed_hbm.at[pl.ds(base + c * C, C)], nsem[b])

    for b in range(NBUF):
        gather(b, b).start()
    pos_cp.wait()
    g_cp.wait()
    b_cp.wait()

    gamma_regs = [gamma_v[pl.ds(16 * j, 16)] for j in range(NREG)]
    beta_regs = [beta_v[pl.ds(16 * j, 16)] for j in range(NREG)]

    def compute_chunk(c, b):
        rows_v, norm_v = rows[b], norm[b]
        off = lax.rem(c * C, L)

        @plsc.parallel_loop(0, C, unroll=2)
        def _tok(t):
            p = off + t
            p = jnp.where(p >= L, p - L, p)
            x = [rows_v[t, pl.ds(16 * j, 16)] + pos_v[p, pl.ds(16 * j, 16)]
                 for j in range(NREG)]
            s = jnp.sum(_tree_sum(x))
            q = jnp.sum(_tree_sum([xj * xj for xj in x]))
            mean = s * (1.0 / DIM)
            var = q * (1.0 / DIM) - mean * mean
            r = _rsqrt(var + EPS)
            a = [r * g for g in gamma_regs]
            for j in range(NREG):
                norm_v[t, pl.ds(16 * j, 16)] = (
                    (x[j] - mean) * a[j] + beta_regs[j])

    def group_body(g, carry):
        for b in range(NBUF):
            c = g * NBUF + b
            gather(c, b).wait()
            words_out(c, b).start()

            @pl.when(g >= 1)
            def _():
                norm_out(c - NBUF, b).wait()

            compute_chunk(c, b)
            norm_out(c, b).start()

            @pl.when(g < NGROUP - 1)
            def _():
                words_out(c, b).wait()
                gather(c + NBUF, b).start()
        return carry

    lax.fori_loop(0, NGROUP, group_body, 0)

    for b in range(NBUF):
        words_out(NCHUNK - NBUF + b, b).wait()
        norm_out(NCHUNK - NBUF + b, b).wait()


@functools.partial(jax.jit, static_argnames=())
def kernel(input_ids, word_table, pos_table, gamma, beta):
    ids_flat = input_ids.reshape(N).astype(jnp.int32)
    mesh = plsc.VectorSubcoreMesh(core_axis_name="c", subcore_axis_name="s")
    normed, words = pl.kernel(
        _sc_body,
        out_type=[
            jax.ShapeDtypeStruct((N, DIM), jnp.float32),
            jax.ShapeDtypeStruct((N, DIM), jnp.float32),
        ],
        mesh=mesh,
        compiler_params=pltpu.CompilerParams(needs_layout_passes=False),
        scratch_types=(
            [
                pltpu.VMEM((TOK_W,), jnp.int32),
                pltpu.VMEM((L, DIM), jnp.float32),
                pltpu.VMEM((DIM,), jnp.float32),
                pltpu.VMEM((DIM,), jnp.float32),
            ]
            + [pltpu.VMEM((C, DIM), jnp.float32)] * (2 * NBUF)
            + [pltpu.SemaphoreType.DMA] * (3 * NBUF)
        ),
    )(ids_flat, word_table, pos_table, gamma, beta)
    return (normed.reshape(B, L, DIM), words.reshape(B, L, DIM))


# speedup vs baseline: 1.3412x; 1.0043x over previous
import functools

import jax
import jax.numpy as jnp
from jax import lax
from jax.experimental import pallas as pl
from jax.experimental.pallas import tpu as pltpu
from jax.experimental.pallas import tpu_sc as plsc

B = 1024
L = 200
DIM = 128
EPS = 1e-12
N = B * L

_info = plsc.get_sparse_core_info()
NC, NS = _info.num_cores, _info.num_subcores
NW = NC * NS
TOK_W = N // NW
C = 80
NCHUNK = TOK_W // C
NBUF = 4
NGROUP = NCHUNK // NBUF
NREG = DIM // 16


def _rsqrt(x):
    i = lax.bitcast_convert_type(x, jnp.int32)
    i = jnp.int32(0x5F3759DF) - (i >> 1)
    y = lax.bitcast_convert_type(i, jnp.float32)
    for _ in range(3):
        y = y * (1.5 - 0.5 * x * y * y)
    return y


def _tree_sum(vs):
    while len(vs) > 1:
        vs = [a + b for a, b in zip(vs[::2], vs[1::2])]
    return vs[0]


def _sc_body(ids_hbm, word_hbm, pos_hbm, gamma_hbm, beta_hbm,
             normed_hbm, words_hbm, ids_v, pos_v, gamma_v, beta_v, *scr):
    rows = scr[:NBUF]
    norm = scr[NBUF:2 * NBUF]
    gsem = scr[2 * NBUF:3 * NBUF]
    wsem = scr[3 * NBUF:4 * NBUF]
    nsem = scr[4 * NBUF:5 * NBUF]

    wid = lax.axis_index("s") * NC + lax.axis_index("c")
    base = wid * TOK_W

    ids_cp = pltpu.async_copy(ids_hbm.at[pl.ds(base, TOK_W)], ids_v, nsem[0])
    pos_cp = pltpu.async_copy(pos_hbm.at[pl.ds(0, L)], pos_v, nsem[1])
    g_cp = pltpu.async_copy(gamma_hbm, gamma_v, nsem[2])
    b_cp = pltpu.async_copy(beta_hbm, beta_v, nsem[3])
    ids_cp.wait()

    def gather(c, b):
        return pltpu.make_async_copy(
            word_hbm.at[ids_v.at[pl.ds(c * C, C)]], rows[b], gsem[b])

    def words_out(c, b):
        return pltpu.make_async_copy(
            rows[b], words_hbm.at[pl.ds(base + c * C, C)], wsem[b])

    def norm_out(c, b):
        return pltpu.make_async_copy(
            norm[b], normed_hbm.at[pl.ds(base + c * C, C)], nsem[b])

    for b in range(NBUF):
        gather(b, b).start()
    pos_cp.wait()
    g_cp.wait()
    b_cp.wait()

    gamma_regs = [gamma_v[pl.ds(16 * j, 16)] for j in range(NREG)]
    beta_regs = [beta_v[pl.ds(16 * j, 16)] for j in range(NREG)]

    def compute_chunk(c, b):
        rows_v, norm_v = rows[b], norm[b]
        off = lax.rem(c * C, L)

        @plsc.parallel_loop(0, C, unroll=2)
        def _tok(t):
            p = off + t
            p = jnp.where(p >= L, p - L, p)
            x = [rows_v[t, pl.ds(16 * j, 16)] + pos_v[p, pl.ds(16 * j, 16)]
                 for j in range(NREG)]
            s = jnp.sum(_tree_sum(x))
            q = jnp.sum(_tree_sum([xj * xj for xj in x]))
            mean = s * (1.0 / DIM)
            var = q * (1.0 / DIM) - mean * mean
            r = _rsqrt(var + EPS)
            a = [r * g for g in gamma_regs]
            for j in range(NREG):
                norm_v[t, pl.ds(16 * j, 16)] = (
                    (x[j] - mean) * a[j] + beta_regs[j])

    def group_body(g, carry):
        for b in range(NBUF):
            c = g * NBUF + b
            gather(c, b).wait()
            words_out(c, b).start()

            @pl.when(g >= 1)
            def _():
                norm_out(c - NBUF, b).wait()

            compute_chunk(c, b)
            norm_out(c, b).start()

            @pl.when(g < NGROUP - 1)
            def _():
                words_out(c, b).wait()
                gather(c + NBUF, b).start()
        return carry

    lax.fori_loop(0, NGROUP, group_body, 0)

    for b in range(NBUF):
        words_out(NCHUNK - NBUF + b, b).wait()
        norm_out(NCHUNK - NBUF + b, b).wait()


@functools.partial(jax.jit, static_argnames=())
def kernel(input_ids, word_table, pos_table, gamma, beta):
    ids_flat = input_ids.reshape(N).astype(jnp.int32)
    mesh = plsc.VectorSubcoreMesh(core_axis_name="c", subcore_axis_name="s")
    normed, words = pl.kernel(
        _sc_body,
        out_type=[
            jax.ShapeDtypeStruct((N, DIM), jnp.float32),
            jax.ShapeDtypeStruct((N, DIM), jnp.float32),
        ],
        mesh=mesh,
        compiler_params=pltpu.CompilerParams(needs_layout_passes=False),
        scratch_types=(
            [
                pltpu.VMEM((TOK_W,), jnp.int32),
                pltpu.VMEM((L, DIM), jnp.float32),
                pltpu.VMEM((DIM,), jnp.float32),
                pltpu.VMEM((DIM,), jnp.float32),
            ]
            + [pltpu.VMEM((C, DIM), jnp.float32)] * (2 * NBUF)
            + [pltpu.SemaphoreType.DMA] * (3 * NBUF)
        ),
    )(ids_flat, word_table, pos_table, gamma, beta)
    return (normed.reshape(B, L, DIM), words.reshape(B, L, DIM))
